# baseline (speedup 1.0000x reference)
import functools

import jax
import jax.numpy as jnp
from jax import lax
from jax.experimental import pallas as pl
from jax.experimental.pallas import tpu as pltpu
from jax.experimental.pallas import tpu_sc as plsc

_NC = 2
_NS = 16
_NW = _NC * _NS

VOCAB = 1000000
DIM = 64
BATCH = 16384
HIST = 50

_LANES = 16
_GRP = 128
_NBLK = VOCAB // _GRP
_VTAIL = _NBLK * _GRP
_Q = VOCAB // 2
_QTAIL = _VTAIL // 2
_NGRP = BATCH * HIST // _GRP
_GPW = _NGRP // _NW
_GSUP = 8

_mesh = lambda: plsc.VectorSubcoreMesh(core_axis_name="c", subcore_axis_name="s")
_params = lambda: pltpu.CompilerParams(
    use_tc_tiling_on_sc=True, needs_layout_passes=False
)


def _ploop(lo, hi):
    return plsc.parallel_loop(lo, hi, unroll=4)


def _gather(ref, idxs):
    return plsc.load_gather(ref, idxs)


def _wid():
    return lax.axis_index("s") * _NC + lax.axis_index("c")


@functools.lru_cache(maxsize=None)
def _phase_a():
    supers = (_NBLK // _NW + 1 + 1) // 2

    def body(wt_hbm, w2_hbm, src0, src1, dst0, dst1, isem0, isem1, osem0, osem1):
        w = _wid()
        iota = lax.iota(jnp.int32, _LANES)

        def transpose_block(src, dst):
            for half in range(2):
                for c0 in range(0, 64, _LANES):
                    d_vec = iota + c0

                    @_ploop(0, 64)
                    def _(qloc, d_vec=d_vec, half=half, c0=c0, src=src, dst=dst):
                        col = jnp.full((_LANES,), qloc * 2 + half, jnp.int32)
                        val = _gather(src, [d_vec, col])
                        dst[qloc, pl.ds(half * 64 + c0, _LANES)] = val

        def in_copy(b, src, isem):
            return pltpu.make_async_copy(
                wt_hbm.at[pl.ds(0, DIM), pl.ds(b * _GRP, _GRP)], src, isem
            )

        def out_copy(b, dst, osem):
            return pltpu.make_async_copy(
                dst, w2_hbm.at[pl.ds(b * (_GRP // 2), _GRP // 2)], osem
            )

        def blk(s, k):
            return w + (s * 2 + k) * _NW

        @pl.when(blk(0, 0) < _NBLK)
        def _p0():
            in_copy(blk(0, 0), src0, isem0).start()

        @pl.when(blk(0, 1) < _NBLK)
        def _p1():
            in_copy(blk(0, 1), src1, isem1).start()

        def super_fn(s, carry):
            for k, (src, dst, isem, osem) in enumerate(
                ((src0, dst0, isem0, osem0), (src1, dst1, isem1, osem1))
            ):
                b = blk(s, k)

                @pl.when(b < _NBLK)
                def _(b=b, src=src, dst=dst, isem=isem, osem=osem, k=k):
                    in_copy(b, src, isem).wait()

                    @pl.when(s > 0)
                    def _():
                        out_copy(0, dst, osem).wait()

                    transpose_block(src, dst)
                    out_copy(b, dst, osem).start()
                    nb = blk(s + 1, k)

                    @pl.when(nb < _NBLK)
                    def _():
                        in_copy(nb, src, isem).start()

            return carry

        lax.fori_loop(0, supers, super_fn, 0)
        out_copy(0, dst0, osem0).wait()
        out_copy(0, dst1, osem1).wait()

    return pl.kernel(
        body,
        out_type=jax.ShapeDtypeStruct((_Q, 2 * DIM), jnp.float32),
        mesh=_mesh(),
        compiler_params=_params(),
        scratch_types=[
            pltpu.VMEM((DIM, _GRP), jnp.float32),
            pltpu.VMEM((DIM, _GRP), jnp.float32),
            pltpu.VMEM((DIM, _GRP), jnp.float32),
            pltpu.VMEM((DIM, _GRP), jnp.float32),
            pltpu.SemaphoreType.DMA,
            pltpu.SemaphoreType.DMA,
            pltpu.SemaphoreType.DMA,
            pltpu.SemaphoreType.DMA,
        ],
    )


@functools.lru_cache(maxsize=None)
def _phase_b():
    supers = _GPW // _GSUP
    rows_n = _GRP + 32

    def body(w2_hbm, idx_hbm, tail_hbm, o3_hbm,
             ibuf, qb0, qb1, rs0, rs1, pb0, pb1, rows0, rows1, out0, out1,
             tsem, gsem0, gsem1, osem0, osem1):
        w = _wid()
        iota = lax.iota(jnp.int32, _LANES)
        g_base = w * _GPW

        pltpu.sync_copy(tail_hbm, rows0.at[pl.ds(_GRP, 32)])
        pltpu.sync_copy(tail_hbm, rows1.at[pl.ds(_GRP, 32)])

        bufs = ((qb0, rs0, pb0, rows0, out0, gsem0, osem0),
                (qb1, rs1, pb1, rows1, out1, gsem1, osem1))

        def prep_group(g_loc, qb, rs, pb):
            for seg in range(_GRP // _LANES):
                v = ibuf[g_loc, pl.ds(seg * _LANES, _LANES)]
                q = lax.shift_right_logical(v, 1)
                qc = jnp.minimum(q, _QTAIL - 1)
                qb[pl.ds(seg * _LANES, _LANES)] = qc
                j = iota + seg * _LANES
                tailsel = q >= _QTAIL
                rs[pl.ds(seg * _LANES, _LANES)] = jnp.where(
                    tailsel, q - _QTAIL + _GRP, j
                )
                pb[pl.ds(seg * _LANES, _LANES)] = (v & 1) * 64

        def transpose_group(rs, pb, rows, out):
            for seg in range(_GRP // _LANES):
                row_vec = rs[pl.ds(seg * _LANES, _LANES)]
                p64 = pb[pl.ds(seg * _LANES, _LANES)]

                @_ploop(0, DIM)
                def _(d, row_vec=row_vec, p64=p64, rows=rows, out=out, seg=seg):
                    val = _gather(rows, [row_vec, p64 + d])
                    out[d, pl.ds(seg * _LANES, _LANES)] = val

        def out_copy(g, out, osem):
            h = g // (BATCH // _GRP)
            bb = g % (BATCH // _GRP)
            return pltpu.make_async_copy(
                out, o3_hbm.at[h, pl.ds(0, DIM), pl.ds(bb * _GRP, _GRP)], osem
            )

        def gather(qb, rows, gsem):
            return pltpu.make_async_copy(
                w2_hbm.at[qb], rows.at[pl.ds(0, _GRP)], gsem
            )

        def super_fn(s, carry):
            pltpu.sync_copy(idx_hbm.at[w, pl.ds(s * _GSUP, _GSUP)], ibuf)
            for k in range(2):
                qb, rs, pb, rows, out, gsem, osem = bufs[k]
                prep_group(k, qb, rs, pb)
                gather(qb, rows, gsem).start()
            for g_loc in range(_GSUP):
                qb, rs, pb, rows, out, gsem, osem = bufs[g_loc % 2]
                gather(qb, rows, gsem).wait()
                if g_loc >= 2:
                    out_copy(0, out, osem).wait()
                else:
                    @pl.when(s > 0)
                    def _(out=out, osem=osem):
                        out_copy(0, out, osem).wait()
                transpose_group(rs, pb, rows, out)
                out_copy(g_base + s * _GSUP + g_loc, out, osem).start()
                if g_loc + 2 < _GSUP:
                    nqb, nrs, npb, nrows, ngsem = (
                        bufs[g_loc % 2][0], bufs[g_loc % 2][1],
                        bufs[g_loc % 2][2], bufs[g_loc % 2][3],
                        bufs[g_loc % 2][5],
                    )
                    prep_group(g_loc + 2, nqb, nrs, npb)
                    gather(nqb, nrows, ngsem).start()
            return carry

        lax.fori_loop(0, supers, super_fn, 0)
        for k in range(2):
            _, _, _, _, out, _, osem = bufs[k]
            out_copy(0, out, osem).wait()

    return pl.kernel(
        body,
        out_type=jax.ShapeDtypeStruct((HIST, DIM, BATCH), jnp.float32),
        mesh=_mesh(),
        compiler_params=_params(),
        scratch_types=[
            pltpu.VMEM((_GSUP, _GRP), jnp.int32),
            pltpu.VMEM((_GRP,), jnp.int32),
            pltpu.VMEM((_GRP,), jnp.int32),
            pltpu.VMEM((_GRP,), jnp.int32),
            pltpu.VMEM((_GRP,), jnp.int32),
            pltpu.VMEM((_GRP,), jnp.int32),
            pltpu.VMEM((_GRP,), jnp.int32),
            pltpu.VMEM((rows_n, 2 * DIM), jnp.float32),
            pltpu.VMEM((rows_n, 2 * DIM), jnp.float32),
            pltpu.VMEM((DIM, _GRP), jnp.float32),
            pltpu.VMEM((DIM, _GRP), jnp.float32),
            pltpu.SemaphoreType.DMA,
            pltpu.SemaphoreType.DMA,
            pltpu.SemaphoreType.DMA,
            pltpu.SemaphoreType.DMA,
            pltpu.SemaphoreType.DMA,
        ],
    )


def kernel(indices, weight):
    assert weight.shape == (VOCAB, DIM) and indices.shape == (BATCH, HIST)
    wt = weight.T
    flat_t = indices.T.reshape(-1).astype(jnp.int32)
    idx3 = flat_t.reshape(_NW, _GPW, _GRP)
    tail = weight[_VTAIL:].reshape(32, 2 * DIM)
    w2 = _phase_a()(wt)
    o3 = _phase_b()(w2, idx3, tail)
    return o3.transpose(2, 0, 1)


# speedup vs baseline: 1.2611x; 1.2611x over previous
import functools

import jax
import jax.numpy as jnp
from jax import lax
from jax.experimental import pallas as pl
from jax.experimental.pallas import tpu as pltpu
from jax.experimental.pallas import tpu_sc as plsc

_INFO = plsc.get_sparse_core_info()
_NC = _INFO.num_cores
_NS = _INFO.num_subcores
_NW = _NC * _NS
_GRP = 128


@functools.lru_cache(maxsize=None)
def _build(vocab: int, dim: int, n_groups: int, k: int):
    rows_per_w = n_groups * _GRP
    steps = n_groups // k
    assert steps % 2 == 0, steps
    supers = steps // 2
    chunk = k * _GRP
    mesh = plsc.VectorSubcoreMesh(core_axis_name="c", subcore_axis_name="s")

    def body(table_hbm, idx_hbm, out_hbm, idx_v, rows0, rows1, gsem0, gsem1,
             wsem0, wsem1):
        wid = lax.axis_index("s") * _NC + lax.axis_index("c")
        base = wid * rows_per_w
        bufs = ((rows0, gsem0, wsem0), (rows1, gsem1, wsem1))

        def super_fn(s, carry):
            g0 = s * 2
            pltpu.sync_copy(idx_hbm.at[wid, pl.ds(g0 * k, 2 * k)], idx_v)
            gathers = []
            for b, (rows, gsem, wsem) in enumerate(bufs):
                @pl.when(s > 0)
                def _drain(rows=rows, wsem=wsem):
                    pltpu.make_async_copy(
                        rows, out_hbm.at[pl.ds(base, chunk)], wsem
                    ).wait()

                gathers.append([
                    pltpu.async_copy(
                        table_hbm.at[idx_v.at[b * k + j]],
                        rows.at[pl.ds(j * _GRP, _GRP)],
                        gsem,
                    )
                    for j in range(k)
                ])
            for b, (rows, gsem, wsem) in enumerate(bufs):
                for c in gathers[b]:
                    c.wait()
                pltpu.make_async_copy(
                    rows,
                    out_hbm.at[pl.ds(base + (g0 + b) * chunk, chunk)],
                    wsem,
                ).start()
            return carry

        lax.fori_loop(0, supers, super_fn, 0)
        for rows, gsem, wsem in bufs:
            pltpu.make_async_copy(
                rows, out_hbm.at[pl.ds(base, chunk)], wsem
            ).wait()

    return pl.kernel(
        body,
        out_type=jax.ShapeDtypeStruct((_NW * rows_per_w, dim), jnp.float32),
        mesh=mesh,
        compiler_params=pltpu.CompilerParams(use_tc_tiling_on_sc=False),
        scratch_types=[
            pltpu.VMEM((2 * k, _GRP), jnp.int32),
            pltpu.VMEM((chunk, dim), jnp.float32),
            pltpu.VMEM((chunk, dim), jnp.float32),
            pltpu.SemaphoreType.DMA,
            pltpu.SemaphoreType.DMA,
            pltpu.SemaphoreType.DMA,
            pltpu.SemaphoreType.DMA,
        ],
    )


def kernel(indices, weight):
    vocab, dim = weight.shape
    out_shape = indices.shape + (dim,)
    flat = indices.reshape(-1).astype(jnp.int32)
    total = flat.shape[0]
    assert total % (_NW * _GRP) == 0, total
    n_groups = total // (_NW * _GRP)
    k = 5
    while n_groups % (2 * k):
        k -= 1
    idx3 = flat.reshape(_NW, n_groups, _GRP)
    out = _build(vocab, dim, n_groups, k)(weight, idx3)
    return out.reshape(out_shape)
